# baseline (speedup 1.0000x reference)
import functools

import jax
import jax.numpy as jnp
import numpy as np
from jax import lax
from jax.experimental import pallas as pl
from jax.experimental.pallas import tpu as pltpu
from jax.experimental.pallas import tpu_sc as plsc

NUM_VOCAB = 1000000
D_MODEL = 64
BATCH = 4096
SEQ = 200
HALF = SEQ // 2
NUM_WORKERS = 32
SEQ_PER_WORKER = BATCH // NUM_WORKERS
SCALE = float(np.sqrt(float(D_MODEL)))


def _position_encoding(max_len, d_model):
    pe = np.zeros((max_len, d_model), dtype=np.float32)
    position = np.arange(0, max_len, dtype=np.float32)[:, None]
    div_term = np.exp(-np.arange(0, d_model, 2, dtype=np.float32)
                      * (np.log(10000.0) / d_model))
    pe[:, 0::2] = np.sin(position * div_term)
    pe[:, 1::2] = np.cos(position * div_term)
    return pe


_PE = _position_encoding(800, D_MODEL)[:SEQ, :]

_mesh = plsc.VectorSubcoreMesh(core_axis_name="c", subcore_axis_name="s")


@functools.partial(
    pl.kernel,
    mesh=_mesh,
    out_type=jax.ShapeDtypeStruct((BATCH, SEQ, D_MODEL), jnp.float32),
    scratch_types=[
        pltpu.VMEM((2, HALF), jnp.int32),
        pltpu.VMEM((SEQ, D_MODEL), jnp.float32),
        pltpu.VMEM((SEQ, D_MODEL), jnp.float32),
        pltpu.SemaphoreType.DMA,
    ],
    compiler_params=pltpu.CompilerParams(use_tc_tiling_on_sc=False),
)
def _emb_lookup(x_hbm, table_hbm, pe_hbm, out_hbm, idx_v, rows_v, pe_v, sem):
    wid = lax.axis_index("s") * 2 + lax.axis_index("c")

    pltpu.sync_copy(pe_hbm, pe_v)

    def seq_body(i, carry):
        seq = wid * SEQ_PER_WORKER + i
        pltpu.sync_copy(x_hbm.at[seq], idx_v)
        cp0 = pltpu.async_copy(
            table_hbm.at[idx_v.at[0]], rows_v.at[pl.ds(0, HALF)], sem)
        cp1 = pltpu.async_copy(
            table_hbm.at[idx_v.at[1]], rows_v.at[pl.ds(HALF, HALF)], sem)
        cp0.wait()
        cp1.wait()

        def comp(p, c):
            for g in range(D_MODEL // 16):
                sl = pl.ds(g * 16, 16)
                rows_v[p, sl] = rows_v[p, sl] * SCALE + pe_v[p, sl]
            return c

        lax.fori_loop(0, SEQ, comp, 0)
        pltpu.sync_copy(rows_v, out_hbm.at[seq])
        return carry

    lax.fori_loop(0, SEQ_PER_WORKER, seq_body, 0)


def kernel(x, table):
    x3 = x.reshape(BATCH, 2, HALF)
    pe = jnp.asarray(_PE)
    return _emb_lookup(x3, table, pe)
